# baseline (speedup 1.0000x reference)
import jax
import jax.numpy as jnp
from jax.experimental import pallas as pl
from jax.experimental.pallas import tpu as pltpu

_N = 512
_OBS = 8
_PRED = 12
_BI = 128
_CB = 128
_NEG = -1e30
_F32 = jnp.float32
_BF16 = jnp.bfloat16
_HI = jax.lax.Precision.HIGHEST


def _cell_t(x_t, h_t, c_t, wih_b, whh_b, bsum):
    g = (jnp.dot(wih_b, x_t.astype(_BF16), preferred_element_type=_F32)
         + jnp.dot(whh_b, h_t.astype(_BF16), preferred_element_type=_F32)
         + bsum)
    gi = jax.nn.sigmoid(g[0:8])
    gf = jax.nn.sigmoid(g[8:16])
    gg = jnp.tanh(g[16:24])
    go = jax.nn.sigmoid(g[24:32])
    c2 = gf * c_t + gi * gg
    h2 = go * jnp.tanh(c2)
    return h2, c2


def _build_body(curr_ref, ci_ref, spw_ref, spb_ref, pr_ref, m01_ref, sb_ref,
                c_ref):
    curr = curr_ref[...]
    ci_blk = ci_ref[...]
    spw = spw_ref[...]
    spb = spb_ref[...]
    pr_b = pr_ref[...]
    m01 = m01_ref[...]
    sb = sb_ref[...]
    for il in range(_CB):
        ci = ci_blk[:, il:il + 1]
        corr = ci - curr
        cb = corr.astype(_BF16).astype(_F32)
        r = (spw[:, 0:1] * cb[0:1, :] + spw[:, 1:2] * cb[1:2, :]) + spb
        t_i = jnp.dot(pr_b, r.astype(_BF16),
                      preferred_element_type=_F32)
        lin = (m01[:, 0:1] * corr[0:1, :] + m01[:, 1:2] * corr[1:2, :] + sb)
        c_ref[il] = (t_i - lin).astype(_BF16)


def _step_body(curr_ref, nei_ref, c_ref, obs_ref, h0_ref, c0_ref, eps_ref,
               enc_w_ref, enc_b_ref, dec_w_ref, dec_b_ref,
               tl_wih_ref, tl_whh_ref, tl_b_ref,
               pl_wih_ref, pl_whh_ref, pl_b_ref,
               m01_ref, bconst_ref, phi_ref, phj_ref,
               p2_w_ref, p2_b_ref,
               mwh_ref, mwc_ref, vwh_ref, vwc_ref, mb_ref, vb_ref,
               preds_ref, means_ref, lvars_ref,
               ph_s, pc_s, ctx_s, out_s):
    t = pl.program_id(0)

    @pl.when(t == 0)
    def _init():
        h = h0_ref[...]
        c = c0_ref[...]
        for k in range(_OBS):
            x = jax.nn.relu(
                jnp.dot(enc_w_ref[...], obs_ref[k],
                        preferred_element_type=_F32) + enc_b_ref[...])
            h, c = _cell_t(x, h, c, tl_wih_ref[...], tl_whh_ref[...],
                           tl_b_ref[...])
        ph_s[...] = h
        pc_s[...] = jnp.zeros_like(c)
        ctx_s[...] = jnp.zeros_like(ctx_s)
        out_s[...] = jnp.zeros_like(out_s)

    inc = jnp.concatenate([ctx_s[...], out_s[...]], axis=0)
    x = jax.nn.relu(
        jnp.dot(dec_w_ref[...], inc.astype(_BF16),
                preferred_element_type=_F32) + dec_b_ref[...])
    ph, pc = _cell_t(x, ph_s[...], pc_s[...], pl_wih_ref[...],
                     pl_whh_ref[...], pl_b_ref[...])
    ph_s[...] = ph
    pc_s[...] = pc

    m01 = m01_ref[...]
    curr = curr_ref[...]
    uu = (m01[:, 0:1] * curr[0:1, :] + m01[:, 1:2] * curr[1:2, :])
    ph_b = ph.astype(_BF16)
    a = uu + jnp.dot(phi_ref[...], ph_b, preferred_element_type=_F32)
    b = (-uu + jnp.dot(phj_ref[...], ph_b, preferred_element_type=_F32)
         + bconst_ref[...])
    w2 = p2_w_ref[...]
    b2 = p2_b_ref[...]
    for ib in range(_N // _BI):
        h1 = jnp.concatenate(
            [jax.nn.relu(a[:, i:i + 1] + b + c_ref[i].astype(_F32))
             .astype(_BF16)
             for i in range(ib * _BI, (ib + 1) * _BI)], axis=1)
        h2 = jnp.dot(w2, h1, preferred_element_type=_F32) + b2
        h2 = h2.reshape(8, _BI, _N)
        mask = nei_ref[0, ib * _BI:(ib + 1) * _BI, :] > 0
        pooled = jnp.max(jnp.where(mask[None], h2, _NEG), axis=2)
        ctx_s[:, ib * _BI:(ib + 1) * _BI] = jax.nn.relu(pooled)

    ctx = ctx_s[...]
    ctx_b = ctx.astype(_BF16)
    mu = (jnp.dot(mwh_ref[...], ph_b, preferred_element_type=_F32)
          + jnp.dot(mwc_ref[...], ctx_b, preferred_element_type=_F32)
          + mb_ref[...])
    lv = (jnp.dot(vwh_ref[...], ph_b, preferred_element_type=_F32)
          + jnp.dot(vwc_ref[...], ctx_b, preferred_element_type=_F32)
          + vb_ref[...])
    out = mu + eps_ref[0] * jnp.exp(0.5 * lv)
    out_s[...] = out
    preds_ref[0] = out
    means_ref[0] = mu
    lvars_ref[0] = lv


def _bf(x):
    return x.astype(_BF16)


def kernel(obs_traj, obs_traj_obs, nei_index, nei_num_index, enc_W, enc_b,
           dec_W, dec_b, tl_Wih, tl_Whh, tl_bih, tl_bhh, pl_Wih, pl_Whh,
           pl_bih, pl_bhh, m_W, m_b, v_W, v_b, sp_W, sp_b, p1_W, p1_b,
           p2_W, p2_b):
    curr_t = obs_traj_obs[-1].T
    kinit = jax.random.key(1)
    h0 = jax.random.normal(jax.random.fold_in(kinit, 0), (_N, 8), _F32).T
    c0 = jax.random.normal(jax.random.fold_in(kinit, 1), (_N, 8), _F32).T
    eps = jnp.stack([
        jax.random.normal(jax.random.fold_in(kinit, 100 + i), (_N, 2), _F32).T
        for i in range(_PRED)])

    p_r = p1_W[:, :32]
    pr_b = _bf(p_r)
    pr_bf = pr_b.astype(_F32)
    spw_bf = _bf(sp_W).astype(_F32)
    m01 = jnp.dot(pr_bf, spw_bf, precision=_HI)
    sb = jnp.dot(sp_b, pr_bf.T, precision=_HI).reshape(64, 1)
    bconst = sb + p1_b.reshape(64, 1)
    mwh = _bf(jnp.concatenate([m_W[:, :4], jnp.zeros((2, 4), _F32)], axis=1))
    mwc = _bf(m_W[:, 4:])
    vwh = _bf(jnp.concatenate([jnp.zeros((2, 4), _F32), v_W[:, :4]], axis=1))
    vwc = _bf(v_W[:, 4:])

    obs_t = _bf(obs_traj.transpose(0, 2, 1))

    c_tab = pl.pallas_call(
        _build_body,
        grid=(_N // _CB,),
        in_specs=[
            pl.BlockSpec((2, _N), lambda i: (0, 0)),
            pl.BlockSpec((2, _CB), lambda i: (0, i)),
            pl.BlockSpec((32, 2), lambda i: (0, 0)),
            pl.BlockSpec((32, 1), lambda i: (0, 0)),
            pl.BlockSpec((64, 32), lambda i: (0, 0)),
            pl.BlockSpec((64, 2), lambda i: (0, 0)),
            pl.BlockSpec((64, 1), lambda i: (0, 0)),
        ],
        out_specs=pl.BlockSpec((_CB, 64, _N), lambda i: (i, 0, 0)),
        out_shape=jax.ShapeDtypeStruct((_N, 64, _N), _BF16),
    )(curr_t, curr_t, spw_bf, sp_b.reshape(32, 1), pr_b, m01, sb)

    def full(shape):
        nd = len(shape)
        return pl.BlockSpec(shape, lambda t, _n=nd: (0,) * _n)

    in_specs = [
        full((2, _N)),
        pl.BlockSpec((1, _N, _N), lambda t: (t, 0, 0)),
        full((_N, 64, _N)),
        full((_OBS, 2, _N)),
        full((8, _N)),
        full((8, _N)),
        pl.BlockSpec((1, 2, _N), lambda t: (t, 0, 0)),
        full((16, 2)), full((16, 1)),
        full((16, 10)), full((16, 1)),
        full((32, 16)), full((32, 8)), full((32, 1)),
        full((32, 16)), full((32, 8)), full((32, 1)),
        full((64, 2)), full((64, 1)),
        full((64, 8)), full((64, 8)),
        full((8, 64)), full((8, 1)),
        full((2, 8)), full((2, 8)), full((2, 8)), full((2, 8)),
        full((2, 1)), full((2, 1)),
    ]
    out_specs = [pl.BlockSpec((1, 2, _N), lambda t: (t, 0, 0))] * 3
    out_shape = [jax.ShapeDtypeStruct((_PRED, 2, _N), _F32)] * 3

    preds_t, means_t, lvars_t = pl.pallas_call(
        _step_body,
        grid=(_PRED,),
        in_specs=in_specs,
        out_specs=out_specs,
        out_shape=out_shape,
        scratch_shapes=[
            pltpu.VMEM((8, _N), _F32),
            pltpu.VMEM((8, _N), _F32),
            pltpu.VMEM((8, _N), _F32),
            pltpu.VMEM((2, _N), _F32),
        ],
    )(curr_t, nei_index, c_tab, obs_t, h0, c0, eps,
      _bf(enc_W), enc_b.reshape(16, 1), _bf(dec_W), dec_b.reshape(16, 1),
      _bf(tl_Wih), _bf(tl_Whh), (tl_bih + tl_bhh).reshape(32, 1),
      _bf(pl_Wih), _bf(pl_Whh), (pl_bih + pl_bhh).reshape(32, 1),
      m01, bconst, _bf(p1_W[:, 40:48]), _bf(p1_W[:, 32:40]),
      _bf(p2_W), p2_b.reshape(8, 1),
      mwh, mwc, vwh, vwc, m_b.reshape(2, 1), v_b.reshape(2, 1))

    return (preds_t.transpose(0, 2, 1), means_t.transpose(0, 2, 1),
            lvars_t.transpose(0, 2, 1))


# speedup vs baseline: 1.0021x; 1.0021x over previous
import jax
import jax.numpy as jnp
from jax.experimental import pallas as pl
from jax.experimental.pallas import tpu as pltpu

_N = 512
_OBS = 8
_PRED = 12
_BI = 128
_CB = 128
_NEG = -1e30
_F32 = jnp.float32
_BF16 = jnp.bfloat16
_HI = jax.lax.Precision.HIGHEST


def _cell_t(x_t, h_t, c_t, wih_b, whh_b, bsum):
    g = (jnp.dot(wih_b, x_t.astype(_BF16), preferred_element_type=_F32)
         + jnp.dot(whh_b, h_t.astype(_BF16), preferred_element_type=_F32)
         + bsum)
    gi = jax.nn.sigmoid(g[0:8])
    gf = jax.nn.sigmoid(g[8:16])
    gg = jnp.tanh(g[16:24])
    go = jax.nn.sigmoid(g[24:32])
    c2 = gf * c_t + gi * gg
    h2 = go * jnp.tanh(c2)
    return h2, c2


def _build_body(curr_ref, ci_ref, spw_ref, spb_ref, pr_ref, m01_ref, sb_ref,
                c_ref):
    curr = curr_ref[...]
    ci_blk = ci_ref[...]
    spw = spw_ref[...]
    spb = spb_ref[...]
    pr_b = pr_ref[...]
    m01 = m01_ref[...]
    sb = sb_ref[...]
    for il in range(_CB):
        ci = ci_blk[:, il:il + 1]
        corr = ci - curr
        cb = corr.astype(_BF16).astype(_F32)
        r = (spw[:, 0:1] * cb[0:1, :] + spw[:, 1:2] * cb[1:2, :]) + spb
        t_i = jnp.dot(pr_b, r.astype(_BF16),
                      preferred_element_type=_F32)
        lin = (m01[:, 0:1] * corr[0:1, :] + m01[:, 1:2] * corr[1:2, :] + sb)
        c_ref[il] = (t_i - lin).astype(_BF16)


def _step_body(curr_ref, nei_ref, c_ref, obs_ref, h0_ref, c0_ref, eps_ref,
               enc_w_ref, enc_b_ref, dec_w_ref, dec_b_ref,
               tl_wih_ref, tl_whh_ref, tl_b_ref,
               pl_wih_ref, pl_whh_ref, pl_b_ref,
               m01_ref, bconst_ref, phi_ref, phj_ref,
               p2_w_ref, p2_b_ref,
               mwh_ref, mwc_ref, vwh_ref, vwc_ref, mb_ref, vb_ref,
               preds_ref, means_ref, lvars_ref,
               ph_s, pc_s, ctx_s, out_s):
    t = pl.program_id(0)

    @pl.when(t == 0)
    def _init():
        h = h0_ref[...]
        c = c0_ref[...]
        for k in range(_OBS):
            x = jax.nn.relu(
                jnp.dot(enc_w_ref[...], obs_ref[k],
                        preferred_element_type=_F32) + enc_b_ref[...])
            h, c = _cell_t(x, h, c, tl_wih_ref[...], tl_whh_ref[...],
                           tl_b_ref[...])
        ph_s[...] = h
        pc_s[...] = jnp.zeros_like(c)
        ctx_s[...] = jnp.zeros_like(ctx_s)
        out_s[...] = jnp.zeros_like(out_s)

    inc = jnp.concatenate([ctx_s[...], out_s[...]], axis=0)
    x = jax.nn.relu(
        jnp.dot(dec_w_ref[...], inc.astype(_BF16),
                preferred_element_type=_F32) + dec_b_ref[...])
    ph, pc = _cell_t(x, ph_s[...], pc_s[...], pl_wih_ref[...],
                     pl_whh_ref[...], pl_b_ref[...])
    ph_s[...] = ph
    pc_s[...] = pc

    uu = jnp.dot(m01_ref[...], curr_ref[...], precision=_HI,
                 preferred_element_type=_F32)
    ph_b = ph.astype(_BF16)
    a = uu + jnp.dot(phi_ref[...], ph_b, preferred_element_type=_F32)
    b = (-uu + jnp.dot(phj_ref[...], ph_b, preferred_element_type=_F32)
         + bconst_ref[...])
    w2 = p2_w_ref[...]
    b2 = p2_b_ref[...]
    for ib in range(_N // _BI):
        h1 = jnp.concatenate(
            [jax.nn.relu(a[:, i:i + 1] + b + c_ref[i].astype(_F32))
             .astype(_BF16)
             for i in range(ib * _BI, (ib + 1) * _BI)], axis=1)
        h2 = jnp.dot(w2, h1, preferred_element_type=_F32) + b2
        h2 = h2.reshape(8, _BI, _N)
        mask = nei_ref[0, ib * _BI:(ib + 1) * _BI, :] > 0
        pooled = jnp.max(jnp.where(mask[None], h2, _NEG), axis=2)
        ctx_s[:, ib * _BI:(ib + 1) * _BI] = jax.nn.relu(pooled)

    ctx = ctx_s[...]
    ctx_b = ctx.astype(_BF16)
    mu = (jnp.dot(mwh_ref[...], ph_b, preferred_element_type=_F32)
          + jnp.dot(mwc_ref[...], ctx_b, preferred_element_type=_F32)
          + mb_ref[...])
    lv = (jnp.dot(vwh_ref[...], ph_b, preferred_element_type=_F32)
          + jnp.dot(vwc_ref[...], ctx_b, preferred_element_type=_F32)
          + vb_ref[...])
    out = mu + eps_ref[0] * jnp.exp(0.5 * lv)
    out_s[...] = out
    preds_ref[0] = out
    means_ref[0] = mu
    lvars_ref[0] = lv


def _bf(x):
    return x.astype(_BF16)


def kernel(obs_traj, obs_traj_obs, nei_index, nei_num_index, enc_W, enc_b,
           dec_W, dec_b, tl_Wih, tl_Whh, tl_bih, tl_bhh, pl_Wih, pl_Whh,
           pl_bih, pl_bhh, m_W, m_b, v_W, v_b, sp_W, sp_b, p1_W, p1_b,
           p2_W, p2_b):
    curr_t = obs_traj_obs[-1].T
    kinit = jax.random.key(1)
    h0 = jax.random.normal(jax.random.fold_in(kinit, 0), (_N, 8), _F32).T
    c0 = jax.random.normal(jax.random.fold_in(kinit, 1), (_N, 8), _F32).T
    eps = jnp.stack([
        jax.random.normal(jax.random.fold_in(kinit, 100 + i), (_N, 2), _F32).T
        for i in range(_PRED)])

    p_r = p1_W[:, :32]
    pr_b = _bf(p_r)
    pr_bf = pr_b.astype(_F32)
    spw_bf = _bf(sp_W).astype(_F32)
    m01 = jnp.dot(pr_bf, spw_bf, precision=_HI)
    sb = jnp.dot(sp_b, pr_bf.T, precision=_HI).reshape(64, 1)
    bconst = sb + p1_b.reshape(64, 1)
    mwh = _bf(jnp.concatenate([m_W[:, :4], jnp.zeros((2, 4), _F32)], axis=1))
    mwc = _bf(m_W[:, 4:])
    vwh = _bf(jnp.concatenate([jnp.zeros((2, 4), _F32), v_W[:, :4]], axis=1))
    vwc = _bf(v_W[:, 4:])

    obs_t = _bf(obs_traj.transpose(0, 2, 1))

    c_tab = pl.pallas_call(
        _build_body,
        grid=(_N // _CB,),
        in_specs=[
            pl.BlockSpec((2, _N), lambda i: (0, 0)),
            pl.BlockSpec((2, _CB), lambda i: (0, i)),
            pl.BlockSpec((32, 2), lambda i: (0, 0)),
            pl.BlockSpec((32, 1), lambda i: (0, 0)),
            pl.BlockSpec((64, 32), lambda i: (0, 0)),
            pl.BlockSpec((64, 2), lambda i: (0, 0)),
            pl.BlockSpec((64, 1), lambda i: (0, 0)),
        ],
        out_specs=pl.BlockSpec((_CB, 64, _N), lambda i: (i, 0, 0)),
        out_shape=jax.ShapeDtypeStruct((_N, 64, _N), _BF16),
    )(curr_t, curr_t, spw_bf, sp_b.reshape(32, 1), pr_b, m01, sb)

    def full(shape):
        nd = len(shape)
        return pl.BlockSpec(shape, lambda t, _n=nd: (0,) * _n)

    in_specs = [
        full((2, _N)),
        pl.BlockSpec((1, _N, _N), lambda t: (t, 0, 0)),
        full((_N, 64, _N)),
        full((_OBS, 2, _N)),
        full((8, _N)),
        full((8, _N)),
        pl.BlockSpec((1, 2, _N), lambda t: (t, 0, 0)),
        full((16, 2)), full((16, 1)),
        full((16, 10)), full((16, 1)),
        full((32, 16)), full((32, 8)), full((32, 1)),
        full((32, 16)), full((32, 8)), full((32, 1)),
        full((64, 2)), full((64, 1)),
        full((64, 8)), full((64, 8)),
        full((8, 64)), full((8, 1)),
        full((2, 8)), full((2, 8)), full((2, 8)), full((2, 8)),
        full((2, 1)), full((2, 1)),
    ]
    out_specs = [pl.BlockSpec((1, 2, _N), lambda t: (t, 0, 0))] * 3
    out_shape = [jax.ShapeDtypeStruct((_PRED, 2, _N), _F32)] * 3

    preds_t, means_t, lvars_t = pl.pallas_call(
        _step_body,
        grid=(_PRED,),
        in_specs=in_specs,
        out_specs=out_specs,
        out_shape=out_shape,
        scratch_shapes=[
            pltpu.VMEM((8, _N), _F32),
            pltpu.VMEM((8, _N), _F32),
            pltpu.VMEM((8, _N), _F32),
            pltpu.VMEM((2, _N), _F32),
        ],
    )(curr_t, nei_index, c_tab, obs_t, h0, c0, eps,
      _bf(enc_W), enc_b.reshape(16, 1), _bf(dec_W), dec_b.reshape(16, 1),
      _bf(tl_Wih), _bf(tl_Whh), (tl_bih + tl_bhh).reshape(32, 1),
      _bf(pl_Wih), _bf(pl_Whh), (pl_bih + pl_bhh).reshape(32, 1),
      m01, bconst, _bf(p1_W[:, 40:48]), _bf(p1_W[:, 32:40]),
      _bf(p2_W), p2_b.reshape(8, 1),
      mwh, mwc, vwh, vwc, m_b.reshape(2, 1), v_b.reshape(2, 1))

    return (preds_t.transpose(0, 2, 1), means_t.transpose(0, 2, 1),
            lvars_t.transpose(0, 2, 1))
